# Initial kernel scaffold; baseline (speedup 1.0000x reference)
import jax
import jax.numpy as jnp
from jax.experimental import pallas as pl


def kernel(x, weights_map):
    raise NotImplementedError("write your pallas kernel here")


# speedup vs baseline: 6.9079x; 6.9079x over previous
import jax
import jax.numpy as jnp
from jax.experimental import pallas as pl

GRID_W = 64
N_NEURONS = 4096
B = 1024
D = 32


def _som_body(x_ref, w_ref, rc_ref, qe_ref):
    x = x_ref[:, :]
    w = w_ref[:, :]

    xw = jax.lax.dot_general(
        x, w, (((1,), (1,)), ((), ())), preferred_element_type=jnp.float32,
        precision=jax.lax.Precision.HIGHEST,
    )
    xn = jnp.sum(x * x, axis=1, keepdims=True)
    wn = jnp.sum(w * w, axis=1)
    dist = xn - 2.0 * xw + wn[None, :]

    col = jax.lax.broadcasted_iota(jnp.int32, dist.shape, 1)
    i1 = jnp.argmin(dist, axis=1).astype(jnp.int32)
    masked = jnp.where(col == i1[:, None], jnp.inf, dist)
    i2 = jnp.argmin(masked, axis=1).astype(jnp.int32)

    oh1 = (col == i1[:, None]).astype(jnp.float32)
    oh2 = (col == i2[:, None]).astype(jnp.float32)
    w1 = jnp.dot(oh1, w, preferred_element_type=jnp.float32,
                 precision=jax.lax.Precision.HIGHEST)
    w2 = jnp.dot(oh2, w, preferred_element_type=jnp.float32,
                 precision=jax.lax.Precision.HIGHEST)

    e1 = jnp.sum(jnp.square(x - w1), axis=1)
    e2 = jnp.sum(jnp.square(x - w2), axis=1)
    use2 = (e2 < e1) | ((e2 == e1) & (i2 < i1))
    bmu = jnp.where(use2, i2, i1)
    qe = jnp.sqrt(jnp.where(use2, e2, e1))

    rc_ref[:, 0] = bmu // GRID_W
    rc_ref[:, 1] = bmu % GRID_W
    qe_ref[:, 0] = qe


def kernel(x, weights_map):
    w_flat = jnp.reshape(weights_map, (N_NEURONS, D))
    rc, qe = pl.pallas_call(
        _som_body,
        out_shape=(
            jax.ShapeDtypeStruct((B, 2), jnp.int32),
            jax.ShapeDtypeStruct((B, 1), jnp.float32),
        ),
    )(x, w_flat)
    return rc, qe[:, 0]
